# Initial kernel scaffold; baseline (speedup 1.0000x reference)
import jax
import jax.numpy as jnp
from jax.experimental import pallas as pl


def kernel(x, edge_index, W, b):
    raise NotImplementedError("write your pallas kernel here")


# speedup vs baseline: 48.7887x; 48.7887x over previous
import jax
import jax.numpy as jnp
from jax import lax
from jax.experimental import pallas as pl
from jax.experimental.pallas import tpu as pltpu, tpu_sc as plsc

N_NODES = 10000
N_EDGES = 320000
DIM = 128
COUT = 16

NS = 16
ROWS_PER_TILE = 632
N_PAD = NS * ROWS_PER_TILE
EDGE_CHUNK = 128
CHUNKS_PER_TILE = 157
E_PAD = NS * CHUNKS_PER_TILE * EDGE_CHUNK

MM_GRID = 8
MM_ROWS = N_PAD // MM_GRID


def _mm_body(x_ref, w_ref, o_ref):
    o_ref[...] = jnp.dot(x_ref[...], w_ref[...],
                         preferred_element_type=jnp.float32)


def _matmul(x_pad, W):
    return pl.pallas_call(
        _mm_body,
        grid=(MM_GRID,),
        in_specs=[
            pl.BlockSpec((MM_ROWS, DIM), lambda i: (i, 0)),
            pl.BlockSpec((DIM, COUT), lambda i: (0, 0)),
        ],
        out_specs=pl.BlockSpec((MM_ROWS, COUT), lambda i: (i, 0)),
        out_shape=jax.ShapeDtypeStruct((N_PAD, COUT), jnp.float32),
    )(x_pad, W)


def _norm_body(xw_ref, deg_ref, yw_ref, dinv_ref):
    dinv = lax.rsqrt(deg_ref[...] + 1.0)
    dinv_ref[...] = dinv
    yw_ref[...] = xw_ref[...] * dinv


def _normalize(xw, deg):
    return pl.pallas_call(
        _norm_body,
        grid=(MM_GRID,),
        in_specs=[
            pl.BlockSpec((MM_ROWS, COUT), lambda i: (i, 0)),
            pl.BlockSpec((MM_ROWS, COUT), lambda i: (i, 0)),
        ],
        out_specs=[
            pl.BlockSpec((MM_ROWS, COUT), lambda i: (i, 0)),
            pl.BlockSpec((MM_ROWS, COUT), lambda i: (i, 0)),
        ],
        out_shape=[
            jax.ShapeDtypeStruct((N_PAD, COUT), jnp.float32),
            jax.ShapeDtypeStruct((N_PAD, COUT), jnp.float32),
        ],
    )(xw, deg)


def _softmax_body(acc_ref, dinv_ref, b_ref, o_ref):
    v = acc_ref[...] * dinv_ref[...] + b_ref[...]
    m = jnp.max(v, axis=1, keepdims=True)
    e = jnp.exp(v - m)
    o_ref[...] = e / jnp.sum(e, axis=1, keepdims=True)


def _softmax(acc, dinv, b):
    return pl.pallas_call(
        _softmax_body,
        grid=(MM_GRID,),
        in_specs=[
            pl.BlockSpec((MM_ROWS, COUT), lambda i: (i, 0)),
            pl.BlockSpec((MM_ROWS, COUT), lambda i: (i, 0)),
            pl.BlockSpec((COUT,), lambda i: (0,)),
        ],
        out_specs=pl.BlockSpec((MM_ROWS, COUT), lambda i: (i, 0)),
        out_shape=jax.ShapeDtypeStruct((N_PAD, COUT), jnp.float32),
    )(acc, dinv, b)


def _sc_deg_body(dst_hbm, deg_hbm, deg_s, dst_v, ones_v, z_v):
    t = lax.axis_index("s")
    base = t * ROWS_PER_TILE

    def _zero(i, c):
        z_v[i, :] = jnp.zeros((COUT,), jnp.float32)
        return c
    lax.fori_loop(0, ROWS_PER_TILE, _zero, 0)

    def _ones(i, c):
        ones_v[i, :] = jnp.ones((COUT,), jnp.float32)
        return c
    lax.fori_loop(0, EDGE_CHUNK, _ones, 0)

    pltpu.sync_copy(z_v, deg_s.at[pl.ds(base, ROWS_PER_TILE)])
    pltpu.sync_copy(dst_hbm.at[t], dst_v)
    plsc.subcore_barrier()

    def _deg(j, c):
        pltpu.sync_copy(ones_v, deg_s.at[dst_v.at[j]], add=True)
        return c
    lax.fori_loop(0, CHUNKS_PER_TILE, _deg, 0)
    plsc.subcore_barrier()

    pltpu.sync_copy(deg_s.at[pl.ds(base, ROWS_PER_TILE)], z_v)
    pltpu.sync_copy(z_v, deg_hbm.at[pl.ds(base, ROWS_PER_TILE)])


def _sc_msg_body(yw_hbm, src_hbm, dst_hbm, out_hbm,
                 ywt_s, acc_s, src_v, dst_v, rows_v, a_v):
    t = lax.axis_index("s")
    base = t * ROWS_PER_TILE

    pltpu.sync_copy(yw_hbm.at[pl.ds(base, ROWS_PER_TILE)], a_v)
    pltpu.sync_copy(a_v, ywt_s.at[pl.ds(base, ROWS_PER_TILE)])
    pltpu.sync_copy(a_v, acc_s.at[pl.ds(base, ROWS_PER_TILE)])
    pltpu.sync_copy(src_hbm.at[t], src_v)
    pltpu.sync_copy(dst_hbm.at[t], dst_v)
    plsc.subcore_barrier()

    def _msg(j, c):
        pltpu.sync_copy(ywt_s.at[src_v.at[j]], rows_v)
        pltpu.sync_copy(rows_v, acc_s.at[dst_v.at[j]], add=True)
        return c
    lax.fori_loop(0, CHUNKS_PER_TILE, _msg, 0)
    plsc.subcore_barrier()

    pltpu.sync_copy(acc_s.at[pl.ds(base, ROWS_PER_TILE)], a_v)
    pltpu.sync_copy(a_v, out_hbm.at[pl.ds(base, ROWS_PER_TILE)])


@jax.jit
def kernel(x, edge_index, W, b):
    n = x.shape[0]
    x_pad = jnp.zeros((N_PAD, DIM), jnp.float32).at[:n].set(x)
    xw = _matmul(x_pad, W)

    src = jnp.full((E_PAD,), n, jnp.int32).at[:N_EDGES].set(edge_index[0])
    dst = jnp.full((E_PAD,), n, jnp.int32).at[:N_EDGES].set(edge_index[1])
    src = src.reshape(NS, CHUNKS_PER_TILE, EDGE_CHUNK)
    dst = dst.reshape(NS, CHUNKS_PER_TILE, EDGE_CHUNK)

    mesh = plsc.VectorSubcoreMesh(core_axis_name="c", subcore_axis_name="s",
                                  num_cores=1)
    sc_params = pltpu.CompilerParams(use_tc_tiling_on_sc=False)

    deg_kernel = pl.kernel(
        _sc_deg_body,
        out_type=jax.ShapeDtypeStruct((N_PAD, COUT), jnp.float32),
        mesh=mesh,
        compiler_params=sc_params,
        scratch_types=[
            pltpu.VMEM_SHARED((N_PAD, COUT), jnp.float32),
            pltpu.VMEM((CHUNKS_PER_TILE, EDGE_CHUNK), jnp.int32),
            pltpu.VMEM((EDGE_CHUNK, COUT), jnp.float32),
            pltpu.VMEM((ROWS_PER_TILE, COUT), jnp.float32),
        ],
    )
    deg = deg_kernel(dst)

    yw, dinv = _normalize(xw, deg)

    msg_kernel = pl.kernel(
        _sc_msg_body,
        out_type=jax.ShapeDtypeStruct((N_PAD, COUT), jnp.float32),
        mesh=mesh,
        compiler_params=sc_params,
        scratch_types=[
            pltpu.VMEM_SHARED((N_PAD, COUT), jnp.float32),
            pltpu.VMEM_SHARED((N_PAD, COUT), jnp.float32),
            pltpu.VMEM((CHUNKS_PER_TILE, EDGE_CHUNK), jnp.int32),
            pltpu.VMEM((CHUNKS_PER_TILE, EDGE_CHUNK), jnp.int32),
            pltpu.VMEM((EDGE_CHUNK, COUT), jnp.float32),
            pltpu.VMEM((ROWS_PER_TILE, COUT), jnp.float32),
        ],
    )
    acc = msg_kernel(yw, src, dst)
    out = _softmax(acc, dinv, b)
    return out[:n]


# speedup vs baseline: 64.5685x; 1.3234x over previous
import jax
import jax.numpy as jnp
from jax import lax
from jax.experimental import pallas as pl
from jax.experimental.pallas import tpu as pltpu, tpu_sc as plsc

N_NODES = 10000
N_EDGES = 320000
DIM = 128
COUT = 16

NS = 16
ROWS_PER_TILE = 640
N_PAD = NS * ROWS_PER_TILE
EDGE_CHUNK = 128
CHUNKS_PER_TILE = 157
E_PAD = NS * CHUNKS_PER_TILE * EDGE_CHUNK

MM_GRID = 8
MM_ROWS = N_PAD // MM_GRID


def _mm_body(x_ref, w_ref, o_ref):
    o_ref[...] = jnp.dot(x_ref[...], w_ref[...],
                         preferred_element_type=jnp.float32)


def _matmul(x_pad, W):
    return pl.pallas_call(
        _mm_body,
        grid=(MM_GRID,),
        in_specs=[
            pl.BlockSpec((MM_ROWS, DIM), lambda i: (i, 0)),
            pl.BlockSpec((DIM, COUT), lambda i: (0, 0)),
        ],
        out_specs=pl.BlockSpec((MM_ROWS, COUT), lambda i: (i, 0)),
        out_shape=jax.ShapeDtypeStruct((N_PAD, COUT), jnp.float32),
    )(x_pad, W)


def _norm_body(xw_ref, deg_ref, yw_ref, dinv_ref):
    i = pl.program_id(0)
    deg = deg_ref[pl.ds(i * MM_ROWS, MM_ROWS)]
    dinv = lax.rsqrt(deg + 1.0)[:, None]
    dinv_ref[...] = jnp.broadcast_to(dinv, (MM_ROWS, COUT))
    yw_ref[...] = xw_ref[...] * dinv


def _normalize(xw, deg):
    return pl.pallas_call(
        _norm_body,
        grid=(MM_GRID,),
        in_specs=[
            pl.BlockSpec((MM_ROWS, COUT), lambda i: (i, 0)),
            pl.BlockSpec((N_PAD,), lambda i: (0,)),
        ],
        out_specs=[
            pl.BlockSpec((MM_ROWS, COUT), lambda i: (i, 0)),
            pl.BlockSpec((MM_ROWS, COUT), lambda i: (i, 0)),
        ],
        out_shape=[
            jax.ShapeDtypeStruct((N_PAD, COUT), jnp.float32),
            jax.ShapeDtypeStruct((N_PAD, COUT), jnp.float32),
        ],
    )(xw, deg)


def _softmax_body(acc_ref, dinv_ref, b_ref, o_ref):
    v = acc_ref[...] * dinv_ref[...] + b_ref[...]
    m = jnp.max(v, axis=1, keepdims=True)
    e = jnp.exp(v - m)
    o_ref[...] = e / jnp.sum(e, axis=1, keepdims=True)


def _softmax(acc, dinv, b):
    return pl.pallas_call(
        _softmax_body,
        grid=(MM_GRID,),
        in_specs=[
            pl.BlockSpec((MM_ROWS, COUT), lambda i: (i, 0)),
            pl.BlockSpec((MM_ROWS, COUT), lambda i: (i, 0)),
            pl.BlockSpec((COUT,), lambda i: (0,)),
        ],
        out_specs=pl.BlockSpec((MM_ROWS, COUT), lambda i: (i, 0)),
        out_shape=jax.ShapeDtypeStruct((N_PAD, COUT), jnp.float32),
    )(acc, dinv, b)


def _sc_deg_body(dst_hbm, deg_hbm, deg_s, dst_v, ones_v, z_v, sem):
    t = lax.axis_index("s")
    base = t * ROWS_PER_TILE

    def _zero(i, c):
        z_v[pl.ds(i * COUT, COUT)] = jnp.zeros((COUT,), jnp.float32)
        return c
    lax.fori_loop(0, ROWS_PER_TILE // COUT, _zero, 0)

    def _ones(i, c):
        ones_v[pl.ds(i * COUT, COUT)] = jnp.ones((COUT,), jnp.float32)
        return c
    lax.fori_loop(0, EDGE_CHUNK // COUT, _ones, 0)

    pltpu.sync_copy(z_v, deg_s.at[pl.ds(base, ROWS_PER_TILE)])
    pltpu.sync_copy(dst_hbm.at[t], dst_v)
    plsc.subcore_barrier()

    def _deg(j, c):
        pltpu.async_copy(ones_v, deg_s.at[dst_v.at[j]], sem, add=True)
        return c
    lax.fori_loop(0, CHUNKS_PER_TILE, _deg, 0)

    def _drain(j, c):
        pltpu.make_async_copy(ones_v, deg_s.at[dst_v.at[0]], sem).wait()
        return c
    lax.fori_loop(0, CHUNKS_PER_TILE, _drain, 0)
    plsc.subcore_barrier()

    pltpu.sync_copy(deg_s.at[pl.ds(base, ROWS_PER_TILE)], z_v)
    pltpu.sync_copy(z_v, deg_hbm.at[pl.ds(base, ROWS_PER_TILE)])


def _sc_msg_body(yw_hbm, src_hbm, dst_hbm, out_hbm,
                 ywt_s, acc_s, src_v, dst_v, rows_v, a_v, gsem, ssem):
    t = lax.axis_index("s")
    base = t * ROWS_PER_TILE

    pltpu.sync_copy(yw_hbm.at[pl.ds(base, ROWS_PER_TILE)], a_v)
    pltpu.sync_copy(a_v, ywt_s.at[pl.ds(base, ROWS_PER_TILE)])
    pltpu.sync_copy(a_v, acc_s.at[pl.ds(base, ROWS_PER_TILE)])
    pltpu.sync_copy(src_hbm.at[t], src_v)
    pltpu.sync_copy(dst_hbm.at[t], dst_v)
    plsc.subcore_barrier()

    pltpu.async_copy(ywt_s.at[src_v.at[0]], rows_v.at[0], gsem)

    def _msg(j, c):
        b = j % 2
        nb = (j + 1) % 2
        pltpu.make_async_copy(ywt_s.at[src_v.at[j]], rows_v.at[b],
                              gsem).wait()

        @pl.when(j >= 1)
        def _():
            pltpu.make_async_copy(rows_v.at[nb], acc_s.at[dst_v.at[j - 1]],
                                  ssem).wait()

        @pl.when(j + 1 < CHUNKS_PER_TILE)
        def _():
            pltpu.async_copy(ywt_s.at[src_v.at[j + 1]], rows_v.at[nb], gsem)

        pltpu.async_copy(rows_v.at[b], acc_s.at[dst_v.at[j]], ssem, add=True)
        return c
    lax.fori_loop(0, CHUNKS_PER_TILE, _msg, 0)
    pltpu.make_async_copy(rows_v.at[0], acc_s.at[dst_v.at[0]], ssem).wait()
    plsc.subcore_barrier()

    pltpu.sync_copy(acc_s.at[pl.ds(base, ROWS_PER_TILE)], a_v)
    pltpu.sync_copy(a_v, out_hbm.at[pl.ds(base, ROWS_PER_TILE)])


@jax.jit
def kernel(x, edge_index, W, b):
    n = x.shape[0]
    x_pad = jnp.zeros((N_PAD, DIM), jnp.float32).at[:n].set(x)
    xw = _matmul(x_pad, W)

    src = jnp.full((E_PAD,), n, jnp.int32).at[:N_EDGES].set(edge_index[0])
    dst = jnp.full((E_PAD,), n, jnp.int32).at[:N_EDGES].set(edge_index[1])
    src = src.reshape(NS, CHUNKS_PER_TILE, EDGE_CHUNK)
    dst = dst.reshape(NS, CHUNKS_PER_TILE, EDGE_CHUNK)

    mesh = plsc.VectorSubcoreMesh(core_axis_name="c", subcore_axis_name="s",
                                  num_cores=1)
    sc_params = pltpu.CompilerParams(use_tc_tiling_on_sc=False)

    deg_kernel = pl.kernel(
        _sc_deg_body,
        out_type=jax.ShapeDtypeStruct((N_PAD,), jnp.float32),
        mesh=mesh,
        compiler_params=sc_params,
        scratch_types=[
            pltpu.VMEM_SHARED((N_PAD,), jnp.float32),
            pltpu.VMEM((CHUNKS_PER_TILE, EDGE_CHUNK), jnp.int32),
            pltpu.VMEM((EDGE_CHUNK,), jnp.float32),
            pltpu.VMEM((ROWS_PER_TILE,), jnp.float32),
            pltpu.SemaphoreType.DMA,
        ],
    )
    deg = deg_kernel(dst)

    yw, dinv = _normalize(xw, deg)

    msg_kernel = pl.kernel(
        _sc_msg_body,
        out_type=jax.ShapeDtypeStruct((N_PAD, COUT), jnp.float32),
        mesh=mesh,
        compiler_params=sc_params,
        scratch_types=[
            pltpu.VMEM_SHARED((N_PAD, COUT), jnp.float32),
            pltpu.VMEM_SHARED((N_PAD, COUT), jnp.float32),
            pltpu.VMEM((CHUNKS_PER_TILE, EDGE_CHUNK), jnp.int32),
            pltpu.VMEM((CHUNKS_PER_TILE, EDGE_CHUNK), jnp.int32),
            pltpu.VMEM((2, EDGE_CHUNK, COUT), jnp.float32),
            pltpu.VMEM((ROWS_PER_TILE, COUT), jnp.float32),
            pltpu.SemaphoreType.DMA,
            pltpu.SemaphoreType.DMA,
        ],
    )
    acc = msg_kernel(yw, src, dst)
    out = _softmax(acc, dinv, b)
    return out[:n]


# speedup vs baseline: 68.0838x; 1.0544x over previous
import jax
import jax.numpy as jnp
from jax import lax
from jax.experimental import pallas as pl
from jax.experimental.pallas import tpu as pltpu, tpu_sc as plsc

N_NODES = 10000
N_EDGES = 320000
DIM = 128
COUT = 16

NC = 2
NS = 16
NW = NC * NS
ROWS_PER_TILE = 640
N_PAD = NS * ROWS_PER_TILE
EDGE_CHUNK = 128
CHUNKS_PER_WORKER = 79
E_PAD = NW * CHUNKS_PER_WORKER * EDGE_CHUNK

MM_GRID = 8
MM_ROWS = N_PAD // MM_GRID


def _mm_body(x_ref, w_ref, o_ref):
    o_ref[...] = jnp.dot(x_ref[...], w_ref[...],
                         preferred_element_type=jnp.float32)


def _matmul(x_pad, W):
    return pl.pallas_call(
        _mm_body,
        grid=(MM_GRID,),
        in_specs=[
            pl.BlockSpec((MM_ROWS, DIM), lambda i: (i, 0)),
            pl.BlockSpec((DIM, COUT), lambda i: (0, 0)),
        ],
        out_specs=pl.BlockSpec((MM_ROWS, COUT), lambda i: (i, 0)),
        out_shape=jax.ShapeDtypeStruct((N_PAD, COUT), jnp.float32),
    )(x_pad, W)


def _norm_body(xw_ref, deg_ref, yw_ref, dinv_ref):
    i = pl.program_id(0)
    deg = (deg_ref[0, pl.ds(i * MM_ROWS, MM_ROWS)]
           + deg_ref[1, pl.ds(i * MM_ROWS, MM_ROWS)])
    dinv = lax.rsqrt(deg + 1.0)[:, None]
    dinv_ref[...] = jnp.broadcast_to(dinv, (MM_ROWS, COUT))
    yw_ref[...] = xw_ref[...] * dinv


def _normalize(xw, deg):
    return pl.pallas_call(
        _norm_body,
        grid=(MM_GRID,),
        in_specs=[
            pl.BlockSpec((MM_ROWS, COUT), lambda i: (i, 0)),
            pl.BlockSpec((NC, N_PAD), lambda i: (0, 0)),
        ],
        out_specs=[
            pl.BlockSpec((MM_ROWS, COUT), lambda i: (i, 0)),
            pl.BlockSpec((MM_ROWS, COUT), lambda i: (i, 0)),
        ],
        out_shape=[
            jax.ShapeDtypeStruct((N_PAD, COUT), jnp.float32),
            jax.ShapeDtypeStruct((N_PAD, COUT), jnp.float32),
        ],
    )(xw, deg)


def _softmax_body(acc_ref, dinv_ref, b_ref, o_ref):
    v = (acc_ref[0] + acc_ref[1]) * dinv_ref[...] + b_ref[...]
    m = jnp.max(v, axis=1, keepdims=True)
    e = jnp.exp(v - m)
    o_ref[...] = e / jnp.sum(e, axis=1, keepdims=True)


def _softmax(acc, dinv, b):
    return pl.pallas_call(
        _softmax_body,
        grid=(MM_GRID,),
        in_specs=[
            pl.BlockSpec((NC, MM_ROWS, COUT), lambda i: (0, i, 0)),
            pl.BlockSpec((MM_ROWS, COUT), lambda i: (i, 0)),
            pl.BlockSpec((COUT,), lambda i: (0,)),
        ],
        out_specs=pl.BlockSpec((MM_ROWS, COUT), lambda i: (i, 0)),
        out_shape=jax.ShapeDtypeStruct((N_PAD, COUT), jnp.float32),
    )(acc, dinv, b)


def _sc_deg_body(dst_hbm, deg_hbm, deg_s, dst_v, ones_v, z_v, sem):
    c = lax.axis_index("c")
    s = lax.axis_index("s")
    w = c * NS + s
    base = s * ROWS_PER_TILE

    def _zero(i, cy):
        z_v[pl.ds(i * COUT, COUT)] = jnp.zeros((COUT,), jnp.float32)
        return cy
    lax.fori_loop(0, ROWS_PER_TILE // COUT, _zero, 0)

    def _ones(i, cy):
        ones_v[pl.ds(i * COUT, COUT)] = jnp.ones((COUT,), jnp.float32)
        return cy
    lax.fori_loop(0, EDGE_CHUNK // COUT, _ones, 0)

    pltpu.sync_copy(z_v, deg_s.at[pl.ds(base, ROWS_PER_TILE)])
    pltpu.sync_copy(dst_hbm.at[w], dst_v)
    plsc.subcore_barrier()

    def _deg(j, cy):
        pltpu.async_copy(ones_v, deg_s.at[dst_v.at[j]], sem, add=True)
        return cy
    lax.fori_loop(0, CHUNKS_PER_WORKER, _deg, 0)

    def _drain(j, cy):
        pltpu.make_async_copy(ones_v, deg_s.at[dst_v.at[0]], sem).wait()
        return cy
    lax.fori_loop(0, CHUNKS_PER_WORKER, _drain, 0)
    plsc.subcore_barrier()

    pltpu.sync_copy(deg_s.at[pl.ds(base, ROWS_PER_TILE)], z_v)
    pltpu.sync_copy(z_v, deg_hbm.at[c, pl.ds(base, ROWS_PER_TILE)])


def _sc_msg_body(yw_hbm, src_hbm, dst_hbm, out_hbm,
                 ywt_s, acc_s, src_v, dst_v, rows_v, a_v, z_v, gsem, ssem):
    c = lax.axis_index("c")
    s = lax.axis_index("s")
    w = c * NS + s
    base = s * ROWS_PER_TILE

    pltpu.sync_copy(yw_hbm.at[pl.ds(base, ROWS_PER_TILE)], a_v)
    pltpu.sync_copy(a_v, ywt_s.at[pl.ds(base, ROWS_PER_TILE)])

    def _zero(i, cy):
        z_v[i, :] = jnp.zeros((COUT,), jnp.float32)
        return cy
    lax.fori_loop(0, ROWS_PER_TILE, _zero, 0)

    @pl.when(c == 0)
    def _():
        pltpu.sync_copy(a_v, acc_s.at[pl.ds(base, ROWS_PER_TILE)])

    @pl.when(c != 0)
    def _():
        pltpu.sync_copy(z_v, acc_s.at[pl.ds(base, ROWS_PER_TILE)])

    pltpu.sync_copy(src_hbm.at[w], src_v)
    pltpu.sync_copy(dst_hbm.at[w], dst_v)
    plsc.subcore_barrier()

    pltpu.async_copy(ywt_s.at[src_v.at[0]], rows_v.at[0], gsem)

    def _msg(j, cy):
        b = j % 2
        nb = (j + 1) % 2
        pltpu.make_async_copy(ywt_s.at[src_v.at[j]], rows_v.at[b],
                              gsem).wait()

        @pl.when(j >= 1)
        def _():
            pltpu.make_async_copy(rows_v.at[nb], acc_s.at[dst_v.at[j - 1]],
                                  ssem).wait()

        @pl.when(j + 1 < CHUNKS_PER_WORKER)
        def _():
            pltpu.async_copy(ywt_s.at[src_v.at[j + 1]], rows_v.at[nb], gsem)

        pltpu.async_copy(rows_v.at[b], acc_s.at[dst_v.at[j]], ssem, add=True)
        return cy
    lax.fori_loop(0, CHUNKS_PER_WORKER, _msg, 0)
    pltpu.make_async_copy(rows_v.at[0], acc_s.at[dst_v.at[0]], ssem).wait()
    plsc.subcore_barrier()

    pltpu.sync_copy(acc_s.at[pl.ds(base, ROWS_PER_TILE)], a_v)
    pltpu.sync_copy(a_v, out_hbm.at[c, pl.ds(base, ROWS_PER_TILE)])


@jax.jit
def kernel(x, edge_index, W, b):
    n = x.shape[0]
    x_pad = jnp.zeros((N_PAD, DIM), jnp.float32).at[:n].set(x)
    xw = _matmul(x_pad, W)

    src = jnp.full((E_PAD,), n, jnp.int32).at[:N_EDGES].set(edge_index[0])
    dst = jnp.full((E_PAD,), n, jnp.int32).at[:N_EDGES].set(edge_index[1])
    src = src.reshape(NW, CHUNKS_PER_WORKER, EDGE_CHUNK)
    dst = dst.reshape(NW, CHUNKS_PER_WORKER, EDGE_CHUNK)

    mesh = plsc.VectorSubcoreMesh(core_axis_name="c", subcore_axis_name="s",
                                  num_cores=NC)
    sc_params = pltpu.CompilerParams(use_tc_tiling_on_sc=False)

    deg_kernel = pl.kernel(
        _sc_deg_body,
        out_type=jax.ShapeDtypeStruct((NC, N_PAD), jnp.float32),
        mesh=mesh,
        compiler_params=sc_params,
        scratch_types=[
            pltpu.VMEM_SHARED((N_PAD,), jnp.float32),
            pltpu.VMEM((CHUNKS_PER_WORKER, EDGE_CHUNK), jnp.int32),
            pltpu.VMEM((EDGE_CHUNK,), jnp.float32),
            pltpu.VMEM((ROWS_PER_TILE,), jnp.float32),
            pltpu.SemaphoreType.DMA,
        ],
    )
    deg = deg_kernel(dst)

    yw, dinv = _normalize(xw, deg)

    msg_kernel = pl.kernel(
        _sc_msg_body,
        out_type=jax.ShapeDtypeStruct((NC, N_PAD, COUT), jnp.float32),
        mesh=mesh,
        compiler_params=sc_params,
        scratch_types=[
            pltpu.VMEM_SHARED((N_PAD, COUT), jnp.float32),
            pltpu.VMEM_SHARED((N_PAD, COUT), jnp.float32),
            pltpu.VMEM((CHUNKS_PER_WORKER, EDGE_CHUNK), jnp.int32),
            pltpu.VMEM((CHUNKS_PER_WORKER, EDGE_CHUNK), jnp.int32),
            pltpu.VMEM((2, EDGE_CHUNK, COUT), jnp.float32),
            pltpu.VMEM((ROWS_PER_TILE, COUT), jnp.float32),
            pltpu.VMEM((ROWS_PER_TILE, COUT), jnp.float32),
            pltpu.SemaphoreType.DMA,
            pltpu.SemaphoreType.DMA,
        ],
    )
    acc = msg_kernel(yw, src, dst)
    out = _softmax(acc, dinv, b)
    return out[:n]
